# baseline (speedup 1.0000x reference)
import functools

import jax
import jax.numpy as jnp
from jax import lax
from jax.experimental import pallas as pl
from jax.experimental.pallas import tpu as pltpu
from jax.experimental.pallas import tpu_sc as plsc

EPS = 1e-15
MAX_LOGPROB = 50.0
TOL = 0.001

N_NODES = 10000
N_EDGES = 160000
D_FEAT = 256

NC, NS = 2, 16
NW = NC * NS
E_PAD = 163840
ESC = E_PAD // NC
CH = 4096
NCHS = ESC // CH
DW = D_FEAT // 2
WPT = DW // NS
SROW = 64
CROWS = CH // SROW


def _sc_dots_body(zt_hbm, ic_hbm, out_hbm, zt, iba, ibb, pba, pbb,
                  sema, semb, semp):
    cid = lax.axis_index("c")
    sid = lax.axis_index("s")
    zeros16 = jnp.zeros((16,), jnp.float32)

    with jax.named_scope("zstage"):
        pltpu.sync_copy(zt_hbm.at[sid], zt)

    icbase = cid * (NCHS * 2 * CH)
    outbase = cid * ESC

    def start(ch, ib, sem):
        pltpu.async_copy(ic_hbm.at[pl.ds(icbase + ch * 2 * CH, 2 * CH)],
                         ib, sem)

    def drain(ib, sem):
        with jax.named_scope("idxwait"):
            pltpu.make_async_copy(ic_hbm.at[pl.ds(0, 2 * CH)], ib, sem).wait()

    def drain_p(pb):
        with jax.named_scope("pwait"):
            pltpu.make_async_copy(pb, out_hbm.at[sid, pl.ds(0, CH)],
                                  semp).wait()

    def compute(ch, ib, pb):
        with jax.named_scope("groups"):
            @plsc.parallel_loop(0, CH // 16, 1, unroll=4)
            def _group(g):
                goff = g * 16
                n0 = ib[pl.ds(goff, 16)]
                n1 = ib[pl.ds(CH + goff, 16)]
                acc0 = zeros16
                acc1 = zeros16
                for kk in range(WPT):
                    kv = jnp.full((16,), kk, jnp.int32)
                    a = plsc.load_gather(zt, [n0, kv])
                    b = plsc.load_gather(zt, [n1, kv])
                    prod = (plsc.bitcast(a, jnp.bfloat16)
                            * plsc.bitcast(b, jnp.bfloat16))
                    p0, p1 = plsc.unpack(prod,
                                         format=plsc.PackFormat.INTERLEAVED)
                    acc0 = acc0 + p0
                    acc1 = acc1 + p1
                pb[pl.ds(pl.multiple_of(goff, 16), 16)] = acc0 + acc1

        pltpu.async_copy(pb, out_hbm.at[sid, pl.ds(outbase + ch * CH, CH)],
                         semp)

    start(0, iba, sema)

    def pair(c2, carry):
        ch = c2 * 2
        start(ch + 1, ibb, semb)
        drain(iba, sema)

        @pl.when(c2 > 0)
        def _():
            drain_p(pba)

        compute(ch, iba, pba)

        @pl.when(c2 + 1 < NCHS // 2)
        def _():
            start(ch + 2, iba, sema)

        drain(ibb, semb)

        @pl.when(c2 > 0)
        def _():
            drain_p(pbb)

        compute(ch + 1, ibb, pbb)
        return carry

    lax.fori_loop(0, NCHS // 2, pair, 0)
    drain_p(pba)
    drain_p(pbb)


@functools.cache
def _get_sc_dots():
    mesh = plsc.VectorSubcoreMesh(core_axis_name="c", subcore_axis_name="s")
    return pl.kernel(
        _sc_dots_body,
        out_type=jax.ShapeDtypeStruct((NS, E_PAD), jnp.float32),
        mesh=mesh,
        scratch_types=[
            pltpu.VMEM((N_NODES, WPT), jnp.int32),
            pltpu.VMEM((2 * CH,), jnp.int32),
            pltpu.VMEM((2 * CH,), jnp.int32),
            pltpu.VMEM((CH,), jnp.float32),
            pltpu.VMEM((CH,), jnp.float32),
            pltpu.SemaphoreType.DMA,
            pltpu.SemaphoreType.DMA,
            pltpu.SemaphoreType.DMA,
        ],
        compiler_params=pltpu.CompilerParams(
            use_tc_tiling_on_sc=False, needs_layout_passes=False
        ),
    )


def _lgamma(x):
    shift = x * (x + 1.0) * (x + 2.0) * (x + 3.0) * (x + 4.0) * (x + 5.0) \
        * (x + 6.0) * (x + 7.0)
    y = x + 8.0
    yi = 1.0 / y
    y2 = yi * yi
    series = yi * (0.083333333333 + y2 * (-0.002777777778 + y2 * 0.000793650794))
    return (y - 0.5) * jnp.log(y) - y + 0.91893853320467 + series - jnp.log(shift)


def _tc_body(d_ref, i0_ref, i1_ref, x_ref, lp_ref, lg_ref, ln_ref, o_ref):
    e_prec = jnp.exp(lp_ref[0, 0])
    e_gam = jnp.exp(lg_ref[0, 0])
    e_n = jnp.exp(ln_ref[0, 0])
    dfl = jnp.abs(i1_ref[...] - i0_ref[...]).astype(jnp.float32) + 1.0
    diff = jnp.exp(-e_gam * jnp.log(dfl))
    dots = jnp.sum(d_ref[...], axis=0)
    p = 1.0 / (1.0 + jnp.exp(-dots))
    alpha = diff * e_prec + p * e_n + EPS
    beta = (1.0 - diff) * e_prec + (1.0 - p) * e_n + EPS
    x = jnp.clip(x_ref[...], TOL, 1.0 - TOL)
    log_prob = (
        (alpha - 1.0) * jnp.log(x)
        + (beta - 1.0) * jnp.log(1.0 - x)
        - (_lgamma(alpha) + _lgamma(beta) - _lgamma(alpha + beta))
    )
    o_ref[...] = jnp.minimum(-log_prob, MAX_LOGPROB)


_RPAD = E_PAD // 128

_tc_call = pl.pallas_call(
    _tc_body,
    out_shape=jax.ShapeDtypeStruct((_RPAD, 128), jnp.float32),
)


def kernel(z, edge_index, edge_attr, logprecision, loggamma, logN):
    idx0 = edge_index[0]
    idx1 = edge_index[1]
    pad = E_PAD - N_EDGES
    zpad = jnp.zeros((pad,), jnp.int32)
    i0p = jnp.concatenate([idx0, zpad])
    i1p = jnp.concatenate([idx1, zpad])
    ic = jnp.stack([i0p.reshape(NC, NCHS, CH), i1p.reshape(NC, NCHS, CH)],
                   axis=2).reshape(2 * E_PAD)
    xp = jnp.concatenate([edge_attr, jnp.full((pad,), 0.5, jnp.float32)])
    z_packed = lax.bitcast_convert_type(
        z.astype(jnp.bfloat16).reshape(N_NODES, DW, 2), jnp.int32)
    z_tiles = z_packed.reshape(N_NODES, NS, WPT).transpose(1, 0, 2)
    partials = _get_sc_dots()(z_tiles, ic)
    out = _tc_call(
        partials.reshape(NS, _RPAD, 128),
        i0p.reshape(_RPAD, 128),
        i1p.reshape(_RPAD, 128),
        xp.reshape(_RPAD, 128),
        logprecision.reshape(1, 1),
        loggamma.reshape(1, 1),
        logN.reshape(1, 1),
    )
    return out.reshape(E_PAD)[:N_EDGES]


# speedup vs baseline: 1.1768x; 1.1768x over previous
import functools

import jax
import jax.numpy as jnp
from jax import lax
from jax.experimental import pallas as pl
from jax.experimental.pallas import tpu as pltpu
from jax.experimental.pallas import tpu_sc as plsc

EPS = 1e-15
MAX_LOGPROB = 50.0
TOL = 0.001

N_NODES = 10000
N_EDGES = 160000
D_FEAT = 256

NC, NS = 2, 16
NW = NC * NS
E_PAD = 163840
ESC = E_PAD // NC
CH = 4096
NCHS = ESC // CH
DW = D_FEAT // 2
WPT = DW // NS
SROW = 64
CROWS = CH // SROW


def _sc_dots_body(zt_hbm, ic_hbm, out_hbm, zt, iba, ibb, pba, pbb,
                  sema, semb, semp):
    cid = lax.axis_index("c")
    sid = lax.axis_index("s")
    zeros16 = jnp.zeros((16,), jnp.float32)

    with jax.named_scope("zstage"):
        pltpu.sync_copy(zt_hbm.at[sid], zt)

    icbase = cid * (NCHS * 2 * CH)
    outbase = cid * ESC

    def start(ch, ib, sem):
        pltpu.async_copy(ic_hbm.at[pl.ds(icbase + ch * 2 * CH, 2 * CH)],
                         ib, sem)

    def drain(ib, sem):
        with jax.named_scope("idxwait"):
            pltpu.make_async_copy(ic_hbm.at[pl.ds(0, 2 * CH)], ib, sem).wait()

    def drain_p(pb):
        with jax.named_scope("pwait"):
            pltpu.make_async_copy(pb, out_hbm.at[sid, pl.ds(0, CH)],
                                  semp).wait()

    def compute(ch, ib, pb):
        with jax.named_scope("groups"):
            @plsc.parallel_loop(0, CH // 16, 1, unroll=2)
            def _group(g):
                goff = g * 16
                n0 = ib[pl.ds(goff, 16)]
                n1 = ib[pl.ds(CH + goff, 16)]
                acc0 = zeros16
                acc1 = zeros16
                for kk in range(WPT):
                    kv = jnp.full((16,), kk, jnp.int32)
                    a = plsc.load_gather(zt, [n0, kv])
                    b = plsc.load_gather(zt, [n1, kv])
                    prod = (plsc.bitcast(a, jnp.bfloat16)
                            * plsc.bitcast(b, jnp.bfloat16))
                    p0, p1 = plsc.unpack(prod,
                                         format=plsc.PackFormat.INTERLEAVED)
                    acc0 = acc0 + p0
                    acc1 = acc1 + p1
                pb[pl.ds(pl.multiple_of(goff, 16), 16)] = acc0 + acc1

        pltpu.async_copy(pb, out_hbm.at[sid, pl.ds(outbase + ch * CH, CH)],
                         semp)

    start(0, iba, sema)

    def pair(c2, carry):
        ch = c2 * 2
        start(ch + 1, ibb, semb)
        drain(iba, sema)

        @pl.when(c2 > 0)
        def _():
            drain_p(pba)

        compute(ch, iba, pba)

        @pl.when(c2 + 1 < NCHS // 2)
        def _():
            start(ch + 2, iba, sema)

        drain(ibb, semb)

        @pl.when(c2 > 0)
        def _():
            drain_p(pbb)

        compute(ch + 1, ibb, pbb)
        return carry

    lax.fori_loop(0, NCHS // 2, pair, 0)
    drain_p(pba)
    drain_p(pbb)


@functools.cache
def _get_sc_dots():
    mesh = plsc.VectorSubcoreMesh(core_axis_name="c", subcore_axis_name="s")
    return pl.kernel(
        _sc_dots_body,
        out_type=jax.ShapeDtypeStruct((NS, E_PAD), jnp.float32),
        mesh=mesh,
        scratch_types=[
            pltpu.VMEM((N_NODES, WPT), jnp.int32),
            pltpu.VMEM((2 * CH,), jnp.int32),
            pltpu.VMEM((2 * CH,), jnp.int32),
            pltpu.VMEM((CH,), jnp.float32),
            pltpu.VMEM((CH,), jnp.float32),
            pltpu.SemaphoreType.DMA,
            pltpu.SemaphoreType.DMA,
            pltpu.SemaphoreType.DMA,
        ],
        compiler_params=pltpu.CompilerParams(
            use_tc_tiling_on_sc=False, needs_layout_passes=False
        ),
    )


def _lgamma(x):
    shift = x * (x + 1.0) * (x + 2.0) * (x + 3.0) * (x + 4.0) * (x + 5.0) \
        * (x + 6.0) * (x + 7.0)
    y = x + 8.0
    yi = 1.0 / y
    y2 = yi * yi
    series = yi * (0.083333333333 + y2 * (-0.002777777778 + y2 * 0.000793650794))
    return (y - 0.5) * jnp.log(y) - y + 0.91893853320467 + series - jnp.log(shift)


def _tc_body(d_ref, i0_ref, i1_ref, x_ref, lp_ref, lg_ref, ln_ref, o_ref):
    e_prec = jnp.exp(lp_ref[0, 0])
    e_gam = jnp.exp(lg_ref[0, 0])
    e_n = jnp.exp(ln_ref[0, 0])
    dfl = jnp.abs(i1_ref[...] - i0_ref[...]).astype(jnp.float32) + 1.0
    diff = jnp.exp(-e_gam * jnp.log(dfl))
    dots = jnp.sum(d_ref[...], axis=0)
    p = 1.0 / (1.0 + jnp.exp(-dots))
    alpha = diff * e_prec + p * e_n + EPS
    beta = (1.0 - diff) * e_prec + (1.0 - p) * e_n + EPS
    x = jnp.clip(x_ref[...], TOL, 1.0 - TOL)
    log_prob = (
        (alpha - 1.0) * jnp.log(x)
        + (beta - 1.0) * jnp.log(1.0 - x)
        - (_lgamma(alpha) + _lgamma(beta) - _lgamma(alpha + beta))
    )
    o_ref[...] = jnp.minimum(-log_prob, MAX_LOGPROB)


_RPAD = E_PAD // 128

_tc_call = pl.pallas_call(
    _tc_body,
    out_shape=jax.ShapeDtypeStruct((_RPAD, 128), jnp.float32),
)


def kernel(z, edge_index, edge_attr, logprecision, loggamma, logN):
    idx0 = edge_index[0]
    idx1 = edge_index[1]
    pad = E_PAD - N_EDGES
    zpad = jnp.zeros((pad,), jnp.int32)
    i0p = jnp.concatenate([idx0, zpad])
    i1p = jnp.concatenate([idx1, zpad])
    ic = jnp.stack([i0p.reshape(NC, NCHS, CH), i1p.reshape(NC, NCHS, CH)],
                   axis=2).reshape(2 * E_PAD)
    xp = jnp.concatenate([edge_attr, jnp.full((pad,), 0.5, jnp.float32)])
    z_packed = lax.bitcast_convert_type(
        z.astype(jnp.bfloat16).reshape(N_NODES, DW, 2), jnp.int32)
    z_tiles = z_packed.reshape(N_NODES, NS, WPT).transpose(1, 0, 2)
    partials = _get_sc_dots()(z_tiles, ic)
    out = _tc_call(
        partials.reshape(NS, _RPAD, 128),
        i0p.reshape(_RPAD, 128),
        i1p.reshape(_RPAD, 128),
        xp.reshape(_RPAD, 128),
        logprecision.reshape(1, 1),
        loggamma.reshape(1, 1),
        logN.reshape(1, 1),
    )
    return out.reshape(E_PAD)[:N_EDGES]
